# baseline (device time: 17565 ns/iter reference)
import jax
import jax.numpy as jnp
from jax import lax
from jax.experimental import pallas as pl
from jax.experimental.pallas import tpu as pltpu

N_DEV = 4
N_HOP = N_DEV - 1
N_SEG = 8
_ORDER = (0, 4, 1, 5, 2, 6, 3, 7)


def kernel(x, w_mat):
    m, k_shard = x.shape
    _, n = w_mat.shape
    m_per = m // N_DEV
    nq = n // N_SEG

    def body(x_hbm, w_hbm, out_hbm, x_ref, w_ref, res, pp, sb, rb,
             ss, sr, sem_in, sem_out):
        my = lax.axis_index("i")
        left = lax.rem(my + N_DEV - 1, N_DEV)
        right = lax.rem(my + 1, N_DEV)

        cp_x = pltpu.make_async_copy(x_hbm, x_ref, sem_in.at[0])
        cp_w = pltpu.make_async_copy(w_hbm, w_ref, sem_in.at[1])
        cp_x.start()
        cp_w.start()

        def goes_right(q):
            return q < N_SEG // 2

        def send_chunk(q, h):
            d = N_DEV - 1 - h if goes_right(q) else 1 + h
            return lax.rem(my + d, N_DEV)

        def recv_chunk(q, h):
            d = N_DEV - 2 - h if goes_right(q) else 2 + h
            return lax.rem(my + d, N_DEV)

        def pchunk(c, q):
            xs = x_ref[pl.ds(c * m_per, m_per), :]
            return jnp.dot(xs, w_ref[:, q * nq:(q + 1) * nq],
                           preferred_element_type=jnp.float32)

        def slot(q, h):
            return q * N_HOP + h

        def mk(q, h):
            dev = right if goes_right(q) else left
            return pltpu.make_async_remote_copy(
                src_ref=sb.at[slot(q, h)], dst_ref=rb.at[slot(q, h)],
                send_sem=ss.at[slot(q, h)], recv_sem=sr.at[slot(q, h)],
                device_id=(dev,), device_id_type=pl.DeviceIdType.MESH,
            )

        rdma = {(q, h): mk(q, h) for q in range(N_SEG) for h in range(N_HOP)}

        barrier_sem = pltpu.get_barrier_semaphore()
        for nbr in (left, right):
            pl.semaphore_signal(
                barrier_sem, inc=1,
                device_id=(nbr,), device_id_type=pl.DeviceIdType.MESH,
            )
        cp_x.wait()
        cp_w.wait()
        for q in _ORDER:
            sb[slot(q, 0), :, :] = pchunk(send_chunk(q, 0), q).astype(
                jnp.bfloat16
            )
        pl.semaphore_wait(barrier_sem, 2)
        for q in _ORDER:
            rdma[(q, 0)].start()

        for h in range(N_HOP - 1):
            for q in _ORDER:
                pp[slot(q, h), :, :] = pchunk(recv_chunk(q, h), q)

        for q in _ORDER:
            rdma[(q, 0)].wait_recv()
            sb[slot(q, 1), :, :] = (
                pp[slot(q, 0), :, :]
                + rb[slot(q, 0), :, :].astype(jnp.float32)
            ).astype(jnp.bfloat16)
            rdma[(q, 1)].start()

        own = {q: pchunk(my, q) for q in _ORDER}

        for q in _ORDER:
            rdma[(q, 1)].wait_recv()
            sb[slot(q, 2), :, :] = (
                pp[slot(q, 1), :, :]
                + rb[slot(q, 1), :, :].astype(jnp.float32)
            ).astype(jnp.bfloat16)
            rdma[(q, 2)].start()

        out_cp = {}
        for q in _ORDER:
            rdma[(q, N_HOP - 1)].wait_recv()
            res[:, q * nq:(q + 1) * nq] = jnp.maximum(
                rb[slot(q, N_HOP - 1), :, :].astype(jnp.float32) + own[q],
                0.0,
            )
            out_cp[q] = pltpu.make_async_copy(
                res.at[:, pl.ds(q * nq, nq)],
                out_hbm.at[:, pl.ds(q * nq, nq)],
                sem_out.at[q],
            )
            out_cp[q].start()

        for q in range(N_SEG):
            for h in range(N_HOP):
                rdma[(q, h)].wait_send()
        for q in _ORDER:
            out_cp[q].wait()

    n_slots = N_SEG * N_HOP
    return pl.pallas_call(
        body,
        out_shape=jax.ShapeDtypeStruct((m_per, n), jnp.float32),
        in_specs=[
            pl.BlockSpec(memory_space=pltpu.MemorySpace.HBM),
            pl.BlockSpec(memory_space=pltpu.MemorySpace.HBM),
        ],
        out_specs=pl.BlockSpec(memory_space=pltpu.MemorySpace.HBM),
        scratch_shapes=[
            pltpu.VMEM((m, k_shard), jnp.float32),
            pltpu.VMEM((k_shard, n), jnp.float32),
            pltpu.VMEM((m_per, n), jnp.float32),
            pltpu.VMEM((N_SEG * (N_HOP - 1), m_per, nq), jnp.float32),
            pltpu.VMEM((n_slots, m_per, nq), jnp.bfloat16),
            pltpu.VMEM((n_slots, m_per, nq), jnp.bfloat16),
            pltpu.SemaphoreType.DMA((n_slots,)),
            pltpu.SemaphoreType.DMA((n_slots,)),
            pltpu.SemaphoreType.DMA((2,)),
            pltpu.SemaphoreType.DMA((N_SEG,)),
        ],
        compiler_params=pltpu.CompilerParams(collective_id=0),
    )(x, w_mat)


# device time: 16694 ns/iter; 1.0522x vs baseline; 1.0522x over previous
import jax
import jax.numpy as jnp
from jax import lax
from jax.experimental import pallas as pl
from jax.experimental.pallas import tpu as pltpu

N_DEV = 4
N_HOP = N_DEV - 1
N_SEG = 8
_ORDER = (0, 4, 1, 5, 2, 6, 3, 7)


def kernel(x, w_mat):
    m, k_shard = x.shape
    _, n = w_mat.shape
    m_per = m // N_DEV
    nq = n // N_SEG

    def body(x_ref, w_ref, out_ref, pp, sb, rb, ss, sr):
        my = lax.axis_index("i")
        left = lax.rem(my + N_DEV - 1, N_DEV)
        right = lax.rem(my + 1, N_DEV)

        def goes_right(q):
            return q < N_SEG // 2

        def send_chunk(q, h):
            d = N_DEV - 1 - h if goes_right(q) else 1 + h
            return lax.rem(my + d, N_DEV)

        def recv_chunk(q, h):
            d = N_DEV - 2 - h if goes_right(q) else 2 + h
            return lax.rem(my + d, N_DEV)

        def pchunk(c, q):
            xs = x_ref[pl.ds(c * m_per, m_per), :]
            return jnp.dot(xs, w_ref[:, q * nq:(q + 1) * nq],
                           preferred_element_type=jnp.float32)

        def slot(q, h):
            return q * N_HOP + h

        def mk(q, h):
            dev = right if goes_right(q) else left
            return pltpu.make_async_remote_copy(
                src_ref=sb.at[slot(q, h)], dst_ref=rb.at[slot(q, h)],
                send_sem=ss.at[slot(q, h)], recv_sem=sr.at[slot(q, h)],
                device_id=(dev,), device_id_type=pl.DeviceIdType.MESH,
            )

        rdma = {(q, h): mk(q, h) for q in range(N_SEG) for h in range(N_HOP)}

        barrier_sem = pltpu.get_barrier_semaphore()
        for nbr in (left, right):
            pl.semaphore_signal(
                barrier_sem, inc=1,
                device_id=(nbr,), device_id_type=pl.DeviceIdType.MESH,
            )
        for q in _ORDER:
            sb[slot(q, 0), :, :] = pchunk(send_chunk(q, 0), q).astype(
                jnp.bfloat16
            )
        pl.semaphore_wait(barrier_sem, 2)
        for q in _ORDER:
            rdma[(q, 0)].start()

        for h in range(N_HOP - 1):
            for q in _ORDER:
                pp[slot(q, h), :, :] = pchunk(recv_chunk(q, h), q)

        for q in _ORDER:
            rdma[(q, 0)].wait_recv()
            sb[slot(q, 1), :, :] = (
                pp[slot(q, 0), :, :]
                + rb[slot(q, 0), :, :].astype(jnp.float32)
            ).astype(jnp.bfloat16)
            rdma[(q, 1)].start()

        own = {q: pchunk(my, q) for q in _ORDER}

        for q in _ORDER:
            rdma[(q, 1)].wait_recv()
            sb[slot(q, 2), :, :] = (
                pp[slot(q, 1), :, :]
                + rb[slot(q, 1), :, :].astype(jnp.float32)
            ).astype(jnp.bfloat16)
            rdma[(q, 2)].start()

        for q in _ORDER:
            rdma[(q, N_HOP - 1)].wait_recv()
            out_ref[:, q * nq:(q + 1) * nq] = jnp.maximum(
                rb[slot(q, N_HOP - 1), :, :].astype(jnp.float32) + own[q],
                0.0,
            )

        for q in range(N_SEG):
            for h in range(N_HOP):
                rdma[(q, h)].wait_send()

    n_slots = N_SEG * N_HOP
    return pl.pallas_call(
        body,
        out_shape=jax.ShapeDtypeStruct((m_per, n), jnp.float32),
        in_specs=[
            pl.BlockSpec(memory_space=pltpu.VMEM),
            pl.BlockSpec(memory_space=pltpu.VMEM),
        ],
        out_specs=pl.BlockSpec(memory_space=pltpu.VMEM),
        scratch_shapes=[
            pltpu.VMEM((N_SEG * (N_HOP - 1), m_per, nq), jnp.float32),
            pltpu.VMEM((n_slots, m_per, nq), jnp.bfloat16),
            pltpu.VMEM((n_slots, m_per, nq), jnp.bfloat16),
            pltpu.SemaphoreType.DMA((n_slots,)),
            pltpu.SemaphoreType.DMA((n_slots,)),
        ],
        compiler_params=pltpu.CompilerParams(collective_id=0),
    )(x, w_mat)
